# baseline (speedup 1.0000x reference)
import functools

import jax
import jax.numpy as jnp
from jax import lax
from jax.experimental import pallas as pl
from jax.experimental.pallas import tpu as pltpu
from jax.experimental.pallas import tpu_sc as plsc

BITVEC = 256
TOK = 16
NTOK = BITVEC // TOK
D = 128
BATCH = 16384
NC, NS = 2, 16
NW = NC * NS
B_PER_W = BATCH // NW
CH = 16
NCHUNK = B_PER_W // CH
TPC = CH * NTOK
G = TPC // 128
XROWS = CH * BITVEC // 128
NXB = 4

_mesh = plsc.VectorSubcoreMesh(
    core_axis_name="c", subcore_axis_name="s", num_cores=NC, num_subcores=NS)


@functools.partial(
    pl.kernel,
    out_type=jax.ShapeDtypeStruct((BATCH, NTOK * D), jnp.float32),
    mesh=_mesh,
    compiler_params=pltpu.CompilerParams(
        needs_layout_passes=False, use_tc_tiling_on_sc=True),
    scratch_types=[
        [pltpu.VMEM((XROWS, 128), jnp.int32)] * NXB,
        [pltpu.VMEM((G, 128), jnp.int32)] * 2,
        [pltpu.VMEM((TPC, D), jnp.float32)] * 2,
        [pltpu.SemaphoreType.DMA] * NXB,
        [pltpu.SemaphoreType.DMA] * 2,
        [pltpu.SemaphoreType.DMA] * 2,
    ],
)
def _emb_kernel(x_hbm, w_hbm, out_hbm, xv, idxv, rowsv, sin, sg, sout):
    wid = lax.axis_index("s") * NC + lax.axis_index("c")
    row0 = wid * B_PER_W
    chunk0 = wid * NCHUNK

    def start_in(i, b):
        return pltpu.async_copy(
            x_hbm.at[pl.ds((chunk0 + i) * XROWS, XROWS), :], xv[b], sin[b])

    def start_out(i, b):
        return pltpu.async_copy(
            rowsv[b].reshape(CH, NTOK * D),
            out_hbm.at[pl.ds(row0 + i * CH, CH), :], sout[b])

    def pack(xb, rb):
        for r in range(CH):
            acc = None
            for k in range(TOK):
                v = xv[xb][2 * r + k // 8, pl.ds((k % 8) * TOK, TOK)]
                term = jnp.left_shift(v, k)
                acc = term if acc is None else acc + term
            idxv[rb][r // 8, pl.ds((r % 8) * TOK, TOK)] = acc

    for p in range(3):
        start_in(p, p)

    def ring_body(j, carry):
        for u in range(NXB):
            i = j * NXB + u
            xb, rb = u, u % 2
            pltpu.make_async_copy(
                x_hbm.at[pl.ds((chunk0 + i) * XROWS, XROWS), :],
                xv[xb], sin[xb]).wait()
            pack(xb, rb)
            @pl.when(i >= 2)
            def _():
                pltpu.make_async_copy(
                    rowsv[rb].reshape(CH, NTOK * D),
                    out_hbm.at[pl.ds(row0 + i * CH, CH), :], sout[rb]).wait()
            for g in range(G):
                pltpu.async_copy(
                    w_hbm.at[idxv[rb].at[g]],
                    rowsv[rb].at[pl.ds(g * 128, 128)], sg[rb])
            @pl.when(i + 3 < NCHUNK)
            def _():
                start_in(i + 3, (u + 3) % NXB)
            pb = 1 - rb
            @pl.when(i >= 1)
            def _():
                for g in range(G):
                    pltpu.make_async_copy(
                        w_hbm.at[idxv[pb].at[g]],
                        rowsv[pb].at[pl.ds(g * 128, 128)], sg[pb]).wait()
                start_out(i - 1, pb)
        return carry

    lax.fori_loop(0, NCHUNK // NXB, ring_body, 0)
    lb = (NCHUNK - 1) % 2
    for g in range(G):
        pltpu.make_async_copy(
            w_hbm.at[idxv[lb].at[g]],
            rowsv[lb].at[pl.ds(g * 128, 128)], sg[lb]).wait()
    start_out(NCHUNK - 1, lb)
    for b in range(2):
        pltpu.make_async_copy(
            rowsv[b].reshape(CH, NTOK * D),
            out_hbm.at[pl.ds(row0, CH), :], sout[b]).wait()


def kernel(x, W):
    xprep = (
        x.reshape(BATCH // CH, CH, NTOK, TOK)
        .transpose(0, 1, 3, 2)
        .reshape(BATCH * BITVEC // 128, 128)
    )
    return _emb_kernel(xprep, W)


# speedup vs baseline: 2.5541x; 2.5541x over previous
import functools

import jax
import jax.numpy as jnp
from jax import lax
from jax.experimental import pallas as pl
from jax.experimental.pallas import tpu as pltpu
from jax.experimental.pallas import tpu_sc as plsc

BITVEC = 256
TOK = 16
NTOK = BITVEC // TOK
D = 128
BATCH = 16384
NC, NS = 2, 16
NW = NC * NS
B_PER_W = BATCH // NW
CH = 8
NCHUNK = B_PER_W // CH
TPC = CH * NTOK
NBUF = 4

_mesh = plsc.VectorSubcoreMesh(
    core_axis_name="c", subcore_axis_name="s", num_cores=NC, num_subcores=NS)


@functools.partial(
    pl.kernel,
    out_type=jax.ShapeDtypeStruct((BATCH, NTOK * D), jnp.float32),
    mesh=_mesh,
    compiler_params=pltpu.CompilerParams(
        needs_layout_passes=False, use_tc_tiling_on_sc=True),
    scratch_types=[
        [pltpu.VMEM((CH, BITVEC), jnp.int32)] * NBUF,
        [pltpu.VMEM((TPC,), jnp.int32)] * NBUF,
        [pltpu.VMEM((TPC, D), jnp.float32)] * NBUF,
        [pltpu.SemaphoreType.DMA] * NBUF,
        [pltpu.SemaphoreType.DMA] * NBUF,
        [pltpu.SemaphoreType.DMA] * NBUF,
    ],
)
def _emb_kernel(x_hbm, w_hbm, out_hbm, xv, idxv, rowsv, sin, sg, sout):
    wid = lax.axis_index("s") * NC + lax.axis_index("c")
    row0 = wid * B_PER_W
    lane = lax.iota(jnp.int32, 16)
    pow2 = jnp.left_shift(jnp.int32(1), lane)

    def start_in(i, b):
        return pltpu.async_copy(
            x_hbm.at[pl.ds(row0 + i * CH, CH), :], xv[b], sin[b])

    def start_out(i, b):
        return pltpu.async_copy(
            rowsv[b].reshape(CH, NTOK * D),
            out_hbm.at[pl.ds(row0 + i * CH, CH), :], sout[b])

    def pack(b):
        xb = xv[b]
        for r in range(CH):
            vals = jnp.zeros((16,), jnp.int32)
            for t in range(NTOK):
                v = xb[r, pl.ds(t * TOK, TOK)]
                s_tok = jnp.sum(v * pow2)
                vals = jnp.where(lane == t, s_tok, vals)
            idxv[b][pl.ds(r * NTOK, NTOK)] = vals

    for p in range(3):
        start_in(p, p)

    def ring_body(j, carry):
        for b in range(NBUF):
            i = j * NBUF + b
            pltpu.make_async_copy(
                x_hbm.at[pl.ds(row0 + i * CH, CH), :], xv[b], sin[b]).wait()
            pack(b)
            @pl.when(i >= NBUF)
            def _():
                pltpu.make_async_copy(
                    rowsv[b].reshape(CH, NTOK * D),
                    out_hbm.at[pl.ds(row0 + i * CH, CH), :], sout[b]).wait()
            pltpu.async_copy(w_hbm.at[idxv[b]], rowsv[b], sg[b])
            @pl.when(i + 3 < NCHUNK)
            def _():
                start_in(i + 3, (b + 3) % NBUF)
            pb = (b - 1) % NBUF
            @pl.when(i >= 1)
            def _():
                pltpu.make_async_copy(
                    w_hbm.at[idxv[pb]], rowsv[pb], sg[pb]).wait()
                start_out(i - 1, pb)
        return carry

    lax.fori_loop(0, NCHUNK // NBUF, ring_body, 0)
    lb = (NCHUNK - 1) % NBUF
    pltpu.make_async_copy(w_hbm.at[idxv[lb]], rowsv[lb], sg[lb]).wait()
    start_out(NCHUNK - 1, lb)
    for b in range(NBUF):
        pltpu.make_async_copy(
            rowsv[b].reshape(CH, NTOK * D),
            out_hbm.at[pl.ds(row0, CH), :], sout[b]).wait()


def kernel(x, W):
    return _emb_kernel(x, W)
